# baseline (device time: 477418 ns/iter reference)
import jax
import jax.numpy as jnp
from jax import lax
from jax.experimental import pallas as pl
from jax.experimental.pallas import tpu as pltpu

N_DEV = 8
M = 4096
K_SH = 512
N = 8192
HOPS = N_DEV - 1

M_BLK = 512
N_BLK = 1024
NT = N // N_BLK
MT = M // M_BLK
W_FRAG = 4

F8 = jnp.float8_e5m2

_RING = [0, 1, 2, 3, 7, 6, 5, 4]
_SUCC = [0] * N_DEV
_PRED = [0] * N_DEV
_RIDX = [0] * N_DEV
for _i, _p in enumerate(_RING):
    _SUCC[_p] = _RING[(_i + 1) % N_DEV]
    _PRED[_p] = _RING[(_i - 1) % N_DEV]
    _RIDX[_p] = _i
_PARTNER = [3, 5, 6, 0, 7, 1, 2, 4]


def kernel(x, w_mat, scale_x, scale_w):
    assert x.shape == (M, K_SH), x.shape
    assert w_mat.shape == (K_SH, N), w_mat.shape

    my = lax.axis_index("i")
    nbrs = jnp.stack([
        jnp.asarray(_SUCC, jnp.int32)[my],
        jnp.asarray(_PRED, jnp.int32)[my],
        jnp.asarray(_PARTNER, jnp.int32)[my],
        jnp.asarray(_RIDX, jnp.int32)[my] % 2,
    ])

    def body(x_ref, w_ref, sx_ref, sw_ref, nbr_ref, out_ref, wg_ref,
             xc_ref, wc_ref, xg_ref, ws_ref, wv_ref, ot_ref, ob_ref,
             conv_sems, xsend, xrecv, wsend, wrecv, wv_sems, out_sems,
             ob_sems):
        right = nbr_ref[0]
        left = nbr_ref[1]
        partner = nbr_ref[2]
        parity = nbr_ref[3]

        xc_ref[...] = x_ref[...].astype(F8)
        for c in range(NT):
            buf = c % 2
            cols = pl.ds(c * N_BLK, N_BLK)
            cp = pltpu.make_async_copy(
                w_ref.at[:, cols], ws_ref.at[buf], conv_sems.at[buf])
            cp.start()
            cp.wait()
            wc_ref[:, cols] = ws_ref[buf].astype(F8)

        barrier = pltpu.get_barrier_semaphore()
        for nbr in (left, right, partner):
            pl.semaphore_signal(barrier, inc=1, device_id=(nbr,),
                                device_id_type=pl.DeviceIdType.MESH)
        pl.semaphore_wait(barrier, 3)

        def remote(xw, src, slot, dev):
            if xw == 0:
                return pltpu.make_async_remote_copy(
                    src_ref=src, dst_ref=xg_ref.at[slot],
                    send_sem=xsend.at[slot], recv_sem=xrecv.at[slot],
                    device_id=(dev,), device_id_type=pl.DeviceIdType.MESH)
            return pltpu.make_async_remote_copy(
                src_ref=src, dst_ref=wg_ref.at[slot],
                send_sem=wsend.at[slot, 0], recv_sem=wrecv.at[slot, 0],
                device_id=(dev,), device_id_type=pl.DeviceIdType.MESH)

        def remote_wfrag(slot, frag, dev):
            fcols = pl.ds(frag * (N // W_FRAG), N // W_FRAG)
            return pltpu.make_async_remote_copy(
                src_ref=wc_ref.at[:, fcols],
                dst_ref=wg_ref.at[slot, :, fcols],
                send_sem=wsend.at[slot, frag], recv_sem=wrecv.at[slot, frag],
                device_id=(dev,), device_id_type=pl.DeviceIdType.MESH)

        xr1 = [remote(0, xc_ref, 0, right),
               remote(0, xc_ref, 2, left),
               remote(0, xc_ref, 4, partner)]
        for r in xr1:
            r.start()
        wr1 = []
        for f in range(W_FRAG):
            for slot, dev in ((0, right), (2, left), (4, partner)):
                wr1.append(remote_wfrag(slot, f, dev))
        for r in wr1:
            r.start()
        for r in xr1:
            r.wait_recv()

        scale = sx_ref[0] * sw_ref[0]
        A_SLOTS = (0, 2, 4)
        B_SLOTS = (1, 3, 5, 6)

        def start_wv(nt, slots):
            buf = nt % 2
            cols = pl.ds(nt * N_BLK, N_BLK)
            cps = [pltpu.make_async_copy(
                wg_ref.at[r, :, cols], wv_ref.at[buf, r], wv_sems.at[buf, r])
                for r in slots]
            for c in cps:
                c.start()
            return cps

        def out_cp(mt, cols):
            rows = pl.ds(mt * M_BLK, M_BLK)
            return pltpu.make_async_copy(
                ot_ref.at[0], out_ref.at[rows, cols], out_sems.at[0])

        frag_waited = set()

        def wait_frag(slot, f, dev):
            if (slot, f) not in frag_waited:
                remote_wfrag(slot, f, dev).wait_recv()
                frag_waited.add((slot, f))

        for nt in range(NT):
            buf = nt % 2
            cols = pl.ds(nt * N_BLK, N_BLK)
            f = nt // (NT // W_FRAG)
            for slot, dev in ((0, right), (2, left), (4, partner)):
                wait_frag(slot, f, dev)
            cps = start_wv(nt, A_SLOTS)
            for c in cps:
                c.wait()

            def mta_body(mt, _, buf=buf, cols=cols, nt=nt):
                rows = pl.ds(mt * M_BLK, M_BLK)
                acc = jnp.dot(xc_ref[rows, :], wc_ref[:, cols],
                              preferred_element_type=jnp.float32)
                for r in A_SLOTS:
                    acc = acc + jnp.dot(xg_ref[r, rows, :], wv_ref[buf, r],
                                        preferred_element_type=jnp.float32)
                @pl.when(jnp.logical_or(mt > 0, nt > 0))
                def _():
                    out_cp(lax.rem(mt + MT - 1, MT), cols).wait()
                ot_ref[0] = acc
                out_cp(mt, cols).start()
                return 0

            lax.fori_loop(0, MT, mta_body, 0)

            if nt == 1:
                for r in xr1:
                    r.wait_send()
                for r in wr1:
                    r.wait_send()
                for ff in range(W_FRAG):
                    wait_frag(0, ff, right)
                    wait_frag(2, ff, left)

                r2 = [remote(0, xg_ref.at[0], 1, right),
                      remote(1, wg_ref.at[0], 1, right),
                      remote(0, xg_ref.at[2], 3, left),
                      remote(1, wg_ref.at[2], 3, left)]
                for r in r2:
                    r.start()

                @pl.when(parity == 0)
                def _():
                    a = remote(0, xg_ref.at[0], 5, partner)
                    b = remote(1, wg_ref.at[0], 5, partner)
                    a.start()
                    b.start()

                @pl.when(parity == 1)
                def _():
                    a = remote(0, xg_ref.at[2], 5, partner)
                    b = remote(1, wg_ref.at[2], 5, partner)
                    a.start()
                    b.start()

            if nt == 5:
                for r in r2:
                    r.wait()
                remote(0, xg_ref.at[0], 5, partner).wait()
                remote(1, wg_ref.at[0], 5, partner).wait()

                @pl.when(parity == 0)
                def _():
                    a = remote(0, xg_ref.at[1], 6, partner)
                    b = remote(1, wg_ref.at[1], 6, partner)
                    a.start()
                    b.start()

                @pl.when(parity == 1)
                def _():
                    a = remote(0, xg_ref.at[3], 6, partner)
                    b = remote(1, wg_ref.at[3], 6, partner)
                    a.start()
                    b.start()

        out_cp(MT - 1, pl.ds((NT - 1) * N_BLK, N_BLK)).wait()
        remote(0, xg_ref.at[1], 6, partner).wait()
        remote(1, wg_ref.at[1], 6, partner).wait()

        def rb_cp(mt, cols, par):
            rows = pl.ds(mt * M_BLK, M_BLK)
            return pltpu.make_async_copy(
                out_ref.at[rows, cols], ob_ref.at[par], ob_sems.at[par])

        pend = start_wv(0, B_SLOTS)
        for nt in range(NT):
            buf = nt % 2
            cols = pl.ds(nt * N_BLK, N_BLK)
            for c in pend:
                c.wait()
            if nt + 1 < NT:
                pend = start_wv(nt + 1, B_SLOTS)

            rb_cp(0, cols, 0).start()
            rb_cp(1, cols, 1).start()

            def mtb_body(i, _, buf=buf, cols=cols, nt=nt):
                for par in (0, 1):
                    mt = i * 2 + par
                    rows = pl.ds(mt * M_BLK, M_BLK)
                    rb_cp(mt, cols, par).wait()
                    acc = ob_ref[par]
                    for r in B_SLOTS:
                        acc = acc + jnp.dot(xg_ref[r, rows, :],
                                            wv_ref[buf, r],
                                            preferred_element_type=jnp.float32)
                    @pl.when(i < MT // 2 - 1)
                    def _():
                        rb_cp(mt + 2, cols, par).start()
                    y = acc * scale
                    val = y / (1.0 + jnp.exp(-jnp.clip(y, -60.0, 60.0)))
                    prev = lax.rem(mt + 2 * MT - 1, MT)
                    if par == 0 and nt == 0:
                        @pl.when(i > 0)
                        def _():
                            out_cp(prev, cols).wait()
                    else:
                        out_cp(prev, cols).wait()
                    ot_ref[0] = val
                    out_cp(mt, cols).start()
                return 0

            lax.fori_loop(0, MT // 2, mtb_body, 0)
        out_cp(MT - 1, pl.ds((NT - 1) * N_BLK, N_BLK)).wait()

    out, _ = pl.pallas_call(
        body,
        out_shape=(jax.ShapeDtypeStruct((M, N), jnp.float32),
                   jax.ShapeDtypeStruct((HOPS, K_SH, N), F8)),
        in_specs=[
            pl.BlockSpec(memory_space=pltpu.VMEM),
            pl.BlockSpec(memory_space=pl.ANY),
            pl.BlockSpec(memory_space=pltpu.SMEM),
            pl.BlockSpec(memory_space=pltpu.SMEM),
            pl.BlockSpec(memory_space=pltpu.SMEM),
        ],
        out_specs=(pl.BlockSpec(memory_space=pl.ANY),
                   pl.BlockSpec(memory_space=pl.ANY)),
        scratch_shapes=[
            pltpu.VMEM((M, K_SH), F8),
            pltpu.VMEM((K_SH, N), F8),
            pltpu.VMEM((HOPS, M, K_SH), F8),
            pltpu.VMEM((2, K_SH, N_BLK), jnp.float32),
            pltpu.VMEM((2, HOPS, K_SH, N_BLK), F8),
            pltpu.VMEM((1, M_BLK, N_BLK), jnp.float32),
            pltpu.VMEM((2, M_BLK, N_BLK), jnp.float32),
            pltpu.SemaphoreType.DMA((2,)),
            pltpu.SemaphoreType.DMA((HOPS,)),
            pltpu.SemaphoreType.DMA((HOPS,)),
            pltpu.SemaphoreType.DMA((HOPS, W_FRAG)),
            pltpu.SemaphoreType.DMA((HOPS, W_FRAG)),
            pltpu.SemaphoreType.DMA((2, HOPS)),
            pltpu.SemaphoreType.DMA((1,)),
            pltpu.SemaphoreType.DMA((2,)),
        ],
        compiler_params=pltpu.CompilerParams(
            collective_id=0,
            vmem_limit_bytes=100 * 1024 * 1024,
        ),
    )(x, w_mat, scale_x, scale_w, nbrs)
    return out


# device time: 474255 ns/iter; 1.0067x vs baseline; 1.0067x over previous
import jax
import jax.numpy as jnp
from jax import lax
from jax.experimental import pallas as pl
from jax.experimental.pallas import tpu as pltpu

N_DEV = 8
M = 4096
K_SH = 512
N = 8192
HOPS = N_DEV - 1

M_BLK = 512
N_BLK = 1024
NT = N // N_BLK
MT = M // M_BLK

F8 = jnp.float8_e5m2

_RING = [0, 1, 2, 3, 7, 6, 5, 4]
_SUCC = [0] * N_DEV
_PRED = [0] * N_DEV
_RIDX = [0] * N_DEV
for _i, _p in enumerate(_RING):
    _SUCC[_p] = _RING[(_i + 1) % N_DEV]
    _PRED[_p] = _RING[(_i - 1) % N_DEV]
    _RIDX[_p] = _i
_PARTNER = [3, 5, 6, 0, 7, 1, 2, 4]


def kernel(x, w_mat, scale_x, scale_w):
    assert x.shape == (M, K_SH), x.shape
    assert w_mat.shape == (K_SH, N), w_mat.shape

    my = lax.axis_index("i")
    nbrs = jnp.stack([
        jnp.asarray(_SUCC, jnp.int32)[my],
        jnp.asarray(_PRED, jnp.int32)[my],
        jnp.asarray(_PARTNER, jnp.int32)[my],
        jnp.asarray(_RIDX, jnp.int32)[my] % 2,
    ])

    def body(x_ref, w_ref, sx_ref, sw_ref, nbr_ref, out_ref, wg_ref,
             xc_ref, wc_ref, xg_ref, ws_ref, wv_ref, ot_ref, ob_ref,
             conv_sems, xsend, xrecv, wsend, wrecv, wv_sems, out_sems,
             ob_sems):
        right = nbr_ref[0]
        left = nbr_ref[1]
        partner = nbr_ref[2]
        parity = nbr_ref[3]

        xc_ref[...] = x_ref[...].astype(F8)
        for c in range(NT):
            buf = c % 2
            cols = pl.ds(c * N_BLK, N_BLK)
            cp = pltpu.make_async_copy(
                w_ref.at[:, cols], ws_ref.at[buf], conv_sems.at[buf])
            cp.start()
            cp.wait()
            wc_ref[:, cols] = ws_ref[buf].astype(F8)

        barrier = pltpu.get_barrier_semaphore()
        for nbr in (left, right, partner):
            pl.semaphore_signal(barrier, inc=1, device_id=(nbr,),
                                device_id_type=pl.DeviceIdType.MESH)
        pl.semaphore_wait(barrier, 3)

        def remote(xw, src, slot, dev):
            g_ref = xg_ref if xw == 0 else wg_ref
            ssem = xsend if xw == 0 else wsend
            rsem = xrecv if xw == 0 else wrecv
            return pltpu.make_async_remote_copy(
                src_ref=src, dst_ref=g_ref.at[slot],
                send_sem=ssem.at[slot], recv_sem=rsem.at[slot],
                device_id=(dev,), device_id_type=pl.DeviceIdType.MESH)

        r1 = [remote(0, xc_ref, 0, right), remote(1, wc_ref, 0, right),
              remote(0, xc_ref, 2, left), remote(1, wc_ref, 2, left),
              remote(0, xc_ref, 4, partner), remote(1, wc_ref, 4, partner)]
        for r in r1:
            r.start()
        for r in r1:
            r.wait()

        r2 = [remote(0, xg_ref.at[0], 1, right), remote(1, wg_ref.at[0], 1, right),
              remote(0, xg_ref.at[2], 3, left), remote(1, wg_ref.at[2], 3, left)]
        for r in r2:
            r.start()

        @pl.when(parity == 0)
        def _():
            a = remote(0, xg_ref.at[0], 5, partner)
            b = remote(1, wg_ref.at[0], 5, partner)
            a.start()
            b.start()

        @pl.when(parity == 1)
        def _():
            a = remote(0, xg_ref.at[2], 5, partner)
            b = remote(1, wg_ref.at[2], 5, partner)
            a.start()
            b.start()


        scale = sx_ref[0] * sw_ref[0]
        A_SLOTS = (0, 2, 4)
        B_SLOTS = (1, 3, 5, 6)

        def start_wv(nt, slots):
            buf = nt % 2
            cols = pl.ds(nt * N_BLK, N_BLK)
            cps = [pltpu.make_async_copy(
                wg_ref.at[r, :, cols], wv_ref.at[buf, r], wv_sems.at[buf, r])
                for r in slots]
            for c in cps:
                c.start()
            return cps

        def out_cp(mt, cols):
            rows = pl.ds(mt * M_BLK, M_BLK)
            return pltpu.make_async_copy(
                ot_ref.at[0], out_ref.at[rows, cols], out_sems.at[0])

        pend = start_wv(0, A_SLOTS)
        for nt in range(NT):
            buf = nt % 2
            cols = pl.ds(nt * N_BLK, N_BLK)
            for c in pend:
                c.wait()
            if nt + 1 < NT:
                pend = start_wv(nt + 1, A_SLOTS)

            def mta_body(mt, _, buf=buf, cols=cols, nt=nt):
                rows = pl.ds(mt * M_BLK, M_BLK)
                acc = jnp.dot(xc_ref[rows, :], wc_ref[:, cols],
                              preferred_element_type=jnp.float32)
                for r in A_SLOTS:
                    acc = acc + jnp.dot(xg_ref[r, rows, :], wv_ref[buf, r],
                                        preferred_element_type=jnp.float32)
                @pl.when(jnp.logical_or(mt > 0, nt > 0))
                def _():
                    out_cp(lax.rem(mt + MT - 1, MT), cols).wait()
                ot_ref[0] = acc
                out_cp(mt, cols).start()
                return 0

            lax.fori_loop(0, MT, mta_body, 0)

            if nt == 2:
                for r in r2:
                    r.wait()
                remote(0, xg_ref.at[0], 5, partner).wait()
                remote(1, wg_ref.at[0], 5, partner).wait()

                @pl.when(parity == 0)
                def _():
                    a = remote(0, xg_ref.at[1], 6, partner)
                    b = remote(1, wg_ref.at[1], 6, partner)
                    a.start()
                    b.start()

                @pl.when(parity == 1)
                def _():
                    a = remote(0, xg_ref.at[3], 6, partner)
                    b = remote(1, wg_ref.at[3], 6, partner)
                    a.start()
                    b.start()

        out_cp(MT - 1, pl.ds((NT - 1) * N_BLK, N_BLK)).wait()
        remote(0, xg_ref.at[1], 6, partner).wait()
        remote(1, wg_ref.at[1], 6, partner).wait()

        def rb_cp(mt, cols, par):
            rows = pl.ds(mt * M_BLK, M_BLK)
            return pltpu.make_async_copy(
                out_ref.at[rows, cols], ob_ref.at[par], ob_sems.at[par])

        pend = start_wv(0, B_SLOTS)
        for nt in range(NT):
            buf = nt % 2
            cols = pl.ds(nt * N_BLK, N_BLK)
            for c in pend:
                c.wait()
            if nt + 1 < NT:
                pend = start_wv(nt + 1, B_SLOTS)

            rb_cp(0, cols, 0).start()
            rb_cp(1, cols, 1).start()

            def mtb_body(i, _, buf=buf, cols=cols, nt=nt):
                for par in (0, 1):
                    mt = i * 2 + par
                    rows = pl.ds(mt * M_BLK, M_BLK)
                    rb_cp(mt, cols, par).wait()
                    acc = ob_ref[par]
                    for r in B_SLOTS:
                        acc = acc + jnp.dot(xg_ref[r, rows, :],
                                            wv_ref[buf, r],
                                            preferred_element_type=jnp.float32)
                    @pl.when(i < MT // 2 - 1)
                    def _():
                        rb_cp(mt + 2, cols, par).start()
                    y = acc * scale
                    val = y / (1.0 + jnp.exp(-y))
                    prev = lax.rem(mt + 2 * MT - 1, MT)
                    if par == 0 and nt == 0:
                        @pl.when(i > 0)
                        def _():
                            out_cp(prev, cols).wait()
                    else:
                        out_cp(prev, cols).wait()
                    ot_ref[0] = val
                    out_cp(mt, cols).start()
                return 0

            lax.fori_loop(0, MT // 2, mtb_body, 0)
        out_cp(MT - 1, pl.ds((NT - 1) * N_BLK, N_BLK)).wait()

    out, _ = pl.pallas_call(
        body,
        out_shape=(jax.ShapeDtypeStruct((M, N), jnp.float32),
                   jax.ShapeDtypeStruct((HOPS, K_SH, N), F8)),
        in_specs=[
            pl.BlockSpec(memory_space=pltpu.VMEM),
            pl.BlockSpec(memory_space=pl.ANY),
            pl.BlockSpec(memory_space=pltpu.SMEM),
            pl.BlockSpec(memory_space=pltpu.SMEM),
            pl.BlockSpec(memory_space=pltpu.SMEM),
        ],
        out_specs=(pl.BlockSpec(memory_space=pl.ANY),
                   pl.BlockSpec(memory_space=pl.ANY)),
        scratch_shapes=[
            pltpu.VMEM((M, K_SH), F8),
            pltpu.VMEM((K_SH, N), F8),
            pltpu.VMEM((HOPS, M, K_SH), F8),
            pltpu.VMEM((2, K_SH, N_BLK), jnp.float32),
            pltpu.VMEM((2, HOPS, K_SH, N_BLK), F8),
            pltpu.VMEM((1, M_BLK, N_BLK), jnp.float32),
            pltpu.VMEM((2, M_BLK, N_BLK), jnp.float32),
            pltpu.SemaphoreType.DMA((2,)),
            pltpu.SemaphoreType.DMA((HOPS,)),
            pltpu.SemaphoreType.DMA((HOPS,)),
            pltpu.SemaphoreType.DMA((HOPS,)),
            pltpu.SemaphoreType.DMA((HOPS,)),
            pltpu.SemaphoreType.DMA((2, HOPS)),
            pltpu.SemaphoreType.DMA((1,)),
            pltpu.SemaphoreType.DMA((2,)),
        ],
        compiler_params=pltpu.CompilerParams(
            collective_id=0,
            vmem_limit_bytes=100 * 1024 * 1024,
        ),
    )(x, w_mat, scale_x, scale_w, nbrs)
    return out
